# baseline (speedup 1.0000x reference)
import functools

import jax
import jax.numpy as jnp
from jax import lax
from jax.experimental import pallas as pl
from jax.experimental.pallas import tpu as pltpu
from jax.experimental.pallas import tpu_sc as plsc

B = 16384
V = 100000
D = 300
DP = 512

_NC = 2
_NS = 16
_NW = _NC * _NS
_ROWS_PER_W = B // _NW
_CH = 128
_NCHUNK = _ROWS_PER_W // _CH


def _sc_gather(src_table, tgt_table, src_id, tgt_id, out,
               idx_v, rows_v, sem):
    wid = lax.axis_index("s") * _NC + lax.axis_index("c")
    base = wid * _ROWS_PER_W
    for c in range(_NCHUNK):
        off = base + c * _CH
        pltpu.sync_copy(src_id.at[pl.ds(off, _CH)], idx_v)
        pltpu.async_copy(src_table.at[idx_v], rows_v, sem).wait()
        pltpu.sync_copy(rows_v, out.at[pl.ds(off, _CH)])
    for c in range(_NCHUNK):
        off = base + c * _CH
        pltpu.sync_copy(tgt_id.at[pl.ds(off, _CH)], idx_v)
        pltpu.async_copy(tgt_table.at[idx_v], rows_v, sem).wait()
        pltpu.sync_copy(rows_v, out.at[pl.ds(B + off, _CH)])


_sc_gather_call = functools.partial(
    pl.kernel,
    out_type=jax.ShapeDtypeStruct((2 * B, DP), jnp.float32),
    mesh=plsc.VectorSubcoreMesh(core_axis_name="c", subcore_axis_name="s"),
    scratch_types=[
        pltpu.VMEM((_CH,), jnp.int32),
        pltpu.VMEM((_CH, DP), jnp.float32),
        pltpu.SemaphoreType.DMA,
    ],
)(_sc_gather)


_BR = 512


_DW = 384


def _tc_body(x_ref, a_ref, be_ref, wd_ref, o_ref):
    i = pl.program_id(0)
    nh = B // _BR

    @pl.when(i < nh)
    def _():
        x = x_ref[...]
        y = lax.dot_general(x, a_ref[...], (((1,), (0,)), ((), ())),
                            preferred_element_type=jnp.float32)
        y = y + be_ref[...]
        o_ref[...] = lax.dot_general(y, wd_ref[...], (((1,), (0,)), ((), ())),
                                     preferred_element_type=jnp.float32)

    @pl.when(i >= nh)
    def _():
        o_ref[...] = x_ref[...][:, :D]


@jax.jit
def kernel(src_table, tgt_table, W_enc, b_enc, W_dec, src_id, tgt_id):
    src_p = jnp.pad(src_table, ((0, 0), (0, DP - D)))
    tgt_p = jnp.pad(tgt_table, ((0, 0), (0, DP - D)))
    gathered = _sc_gather_call(src_p, tgt_p, src_id, tgt_id)
    A = jnp.pad(W_enc.T, ((0, _DW - D), (0, 0)))
    grid = (2 * B) // _BR
    out = pl.pallas_call(
        _tc_body,
        grid=(grid,),
        in_specs=[
            pl.BlockSpec((_BR, _DW), lambda i: (i, 0)),
            pl.BlockSpec((_DW, D), lambda i: (0, 0)),
            pl.BlockSpec((1, D), lambda i: (0, 0)),
            pl.BlockSpec((D, D), lambda i: (0, 0)),
        ],
        out_specs=pl.BlockSpec((_BR, D), lambda i: (i, 0)),
        out_shape=jax.ShapeDtypeStruct((2 * B, D), jnp.float32),
    )(gathered, A, b_enc.reshape(1, D), W_dec)
    return out


# speedup vs baseline: 1.0212x; 1.0212x over previous
import functools

import jax
import jax.numpy as jnp
from jax import lax
from jax.experimental import pallas as pl
from jax.experimental.pallas import tpu as pltpu
from jax.experimental.pallas import tpu_sc as plsc

B = 16384
V = 100000
D = 300
DP = 384

_NC = 2
_NS = 16
_NW = _NC * _NS
_ROWS_PER_W = B // _NW
_CH = 128
_NCHUNK = _ROWS_PER_W // _CH


def _sc_gather(src_table, tgt_table, src_id, tgt_id, out,
               idx_v, rows_v, sem):
    wid = lax.axis_index("s") * _NC + lax.axis_index("c")
    base = wid * _ROWS_PER_W
    for c in range(_NCHUNK):
        off = base + c * _CH
        pltpu.sync_copy(src_id.at[pl.ds(off, _CH)], idx_v)
        pltpu.async_copy(src_table.at[idx_v], rows_v, sem).wait()
        pltpu.sync_copy(rows_v, out.at[pl.ds(off, _CH)])
    for c in range(_NCHUNK):
        off = base + c * _CH
        pltpu.sync_copy(tgt_id.at[pl.ds(off, _CH)], idx_v)
        pltpu.async_copy(tgt_table.at[idx_v], rows_v, sem).wait()
        pltpu.sync_copy(rows_v, out.at[pl.ds(B + off, _CH)])


_sc_gather_call = functools.partial(
    pl.kernel,
    out_type=jax.ShapeDtypeStruct((2 * B, DP), jnp.float32),
    mesh=plsc.VectorSubcoreMesh(core_axis_name="c", subcore_axis_name="s"),
    scratch_types=[
        pltpu.VMEM((_CH,), jnp.int32),
        pltpu.VMEM((_CH, DP), jnp.float32),
        pltpu.SemaphoreType.DMA,
    ],
)(_sc_gather)


_BR = 512


def _tc_body(x_ref, a_ref, be_ref, wd_ref, o_ref):
    i = pl.program_id(0)
    nh = B // _BR

    @pl.when(i < nh)
    def _():
        x = x_ref[...]
        y = lax.dot_general(x, a_ref[...], (((1,), (0,)), ((), ())),
                            preferred_element_type=jnp.float32)
        y = y + be_ref[...]
        o_ref[...] = lax.dot_general(y, wd_ref[...], (((1,), (0,)), ((), ())),
                                     preferred_element_type=jnp.float32)

    @pl.when(i >= nh)
    def _():
        o_ref[...] = x_ref[...][:, :D]


@jax.jit
def kernel(src_table, tgt_table, W_enc, b_enc, W_dec, src_id, tgt_id):
    src_p = jnp.pad(src_table, ((0, 0), (0, DP - D)))
    tgt_p = jnp.pad(tgt_table, ((0, 0), (0, DP - D)))
    gathered = _sc_gather_call(src_p, tgt_p, src_id, tgt_id)
    A = jnp.pad(W_enc.T, ((0, DP - D), (0, 0)))
    grid = (2 * B) // _BR
    out = pl.pallas_call(
        _tc_body,
        grid=(grid,),
        in_specs=[
            pl.BlockSpec((_BR, DP), lambda i: (i, 0)),
            pl.BlockSpec((DP, D), lambda i: (0, 0)),
            pl.BlockSpec((1, D), lambda i: (0, 0)),
            pl.BlockSpec((D, D), lambda i: (0, 0)),
        ],
        out_specs=pl.BlockSpec((_BR, D), lambda i: (i, 0)),
        out_shape=jax.ShapeDtypeStruct((2 * B, D), jnp.float32),
    )(gathered, A, b_enc.reshape(1, D), W_dec)
    return out
